# Initial kernel scaffold; baseline (speedup 1.0000x reference)
import jax
import jax.numpy as jnp
from jax.experimental import pallas as pl


def kernel(h_source, h_target, src_mask, tgt_mask):
    raise NotImplementedError("write your pallas kernel here")


# speedup vs baseline: 1.3698x; 1.3698x over previous
import jax
import jax.numpy as jnp
from jax.experimental import pallas as pl
from jax.experimental.pallas import tpu as pltpu

_TEMPERATURE = 0.07
_ROW_BLOCK = 512


def _tcl_body(hs_ref, ht_ref, ms_ref, mt_ref, out_ref, acc_ref, carry_ref):
    i = pl.program_id(0)
    nb = pl.num_programs(0)
    n = ht_ref.shape[0]
    r = hs_ref.shape[0]

    ht = ht_ref[...] * mt_ref[...]
    tinv = jax.lax.rsqrt(jnp.maximum(jnp.sum(ht * ht, axis=1, keepdims=True),
                                     1e-24))
    htn = ht * tinv

    hs = hs_ref[...] * ms_ref[...]
    sinv = jax.lax.rsqrt(jnp.maximum(jnp.sum(hs * hs, axis=1, keepdims=True),
                                     1e-24))
    hsn = hs * sinv

    sim = jax.lax.dot_general(hsn, htn, (((1,), (1,)), ((), ())),
                              precision=jax.lax.Precision.HIGHEST,
                              preferred_element_type=jnp.float32)

    m = jnp.max(sim, axis=1, keepdims=True)
    iota = jax.lax.broadcasted_iota(jnp.int32, sim.shape, 1)
    amax = jnp.min(jnp.where(sim == m, iota, n), axis=1)

    e = jnp.exp((sim - m) * (1.0 / _TEMPERATURE))
    log_s = jnp.log(jnp.sum(e, axis=1))

    onehot = (iota == amax[:, None]).astype(jnp.float32)
    g = jax.lax.dot_general(onehot, htn, (((1,), (0,)), ((), ())),
                            preferred_element_type=jnp.float32)

    nn_step = jnp.sum(g[: r - 1, :] * g[1:, :])

    @pl.when(i == 0)
    def _init():
        acc_ref[0] = 0.0
        acc_ref[1] = 0.0

    @pl.when(i > 0)
    def _boundary():
        acc_ref[1] += jnp.sum(carry_ref[0, :] * g[0, :])

    acc_ref[0] += jnp.sum(log_s)
    acc_ref[1] += nn_step
    carry_ref[0, :] = g[r - 1, :]

    @pl.when(i == nb - 1)
    def _emit():
        out_ref[0] = acc_ref[0] / n
        out_ref[1] = 1.0 - acc_ref[1] / (n - 1)


def kernel(h_source, h_target, src_mask, tgt_mask):
    b, t, h = h_source.shape
    n = b * t
    r = _ROW_BLOCK
    hs = h_source.reshape(n, h).astype(jnp.float32)
    ht = h_target.reshape(n, h).astype(jnp.float32)
    ms = src_mask.reshape(n, 1).astype(jnp.float32)
    mt = tgt_mask.reshape(n, 1).astype(jnp.float32)

    out = pl.pallas_call(
        _tcl_body,
        grid=(n // r,),
        in_specs=[
            pl.BlockSpec((r, h), lambda i: (i, 0)),
            pl.BlockSpec((n, h), lambda i: (0, 0)),
            pl.BlockSpec((r, 1), lambda i: (i, 0)),
            pl.BlockSpec((n, 1), lambda i: (0, 0)),
        ],
        out_specs=pl.BlockSpec(memory_space=pltpu.SMEM),
        out_shape=jax.ShapeDtypeStruct((2,), jnp.float32),
        scratch_shapes=[
            pltpu.SMEM((2,), jnp.float32),
            pltpu.VMEM((1, h), jnp.float32),
        ],
        compiler_params=pltpu.CompilerParams(
            dimension_semantics=("arbitrary",),
        ),
    )(hs, ht, ms, mt)
    return (out[0], out[1])


# speedup vs baseline: 2.9922x; 2.1844x over previous
import jax
import jax.numpy as jnp
from jax.experimental import pallas as pl
from jax.experimental.pallas import tpu as pltpu

_TEMPERATURE = 0.07
_ROW_BLOCK = 512


def _tcl_body(hs_ref, ht_ref, ms_ref, mt_ref, out_ref, acc_ref, carry_ref,
              htn_ref):
    i = pl.program_id(0)
    nb = pl.num_programs(0)
    n = ht_ref.shape[0]
    r = hs_ref.shape[0]

    @pl.when(i == 0)
    def _prep():
        ht = ht_ref[...] * mt_ref[...]
        tinv = jax.lax.rsqrt(
            jnp.maximum(jnp.sum(ht * ht, axis=1, keepdims=True), 1e-24))
        htn_ref[...] = ht * tinv

    htn = htn_ref[...]

    hs = hs_ref[...] * ms_ref[...]
    sinv = jax.lax.rsqrt(
        jnp.maximum(jnp.sum(hs * hs, axis=1, keepdims=True), 1e-24))
    hsn = hs * (sinv * (1.0 / _TEMPERATURE))

    sim = jax.lax.dot_general(hsn, htn, (((1,), (1,)), ((), ())),
                              preferred_element_type=jnp.float32)

    m = jnp.max(sim, axis=1, keepdims=True)
    iota = jax.lax.broadcasted_iota(jnp.int32, sim.shape, 1)
    amax = jnp.min(jnp.where(sim == m, iota, n), axis=1)

    log_s = jnp.log(jnp.sum(jnp.exp(sim), axis=1)) - m[:, 0]

    onehot = (iota == amax[:, None]).astype(jnp.float32)
    g = jax.lax.dot_general(onehot, htn, (((1,), (0,)), ((), ())),
                            preferred_element_type=jnp.float32)

    nn_step = jnp.sum(g[: r - 1, :] * g[1:, :])

    @pl.when(i == 0)
    def _init():
        acc_ref[0] = 0.0
        acc_ref[1] = 0.0

    @pl.when(i > 0)
    def _boundary():
        acc_ref[1] += jnp.sum(carry_ref[0, :] * g[0, :])

    acc_ref[0] += jnp.sum(log_s)
    acc_ref[1] += nn_step
    carry_ref[0, :] = g[r - 1, :]

    @pl.when(i == nb - 1)
    def _emit():
        out_ref[0] = acc_ref[0] / n
        out_ref[1] = 1.0 - acc_ref[1] / (n - 1)


def kernel(h_source, h_target, src_mask, tgt_mask):
    b, t, h = h_source.shape
    n = b * t
    r = _ROW_BLOCK
    hs = h_source.reshape(n, h).astype(jnp.float32)
    ht = h_target.reshape(n, h).astype(jnp.float32)
    ms = src_mask.reshape(n, 1).astype(jnp.float32)
    mt = tgt_mask.reshape(n, 1).astype(jnp.float32)

    out = pl.pallas_call(
        _tcl_body,
        grid=(n // r,),
        in_specs=[
            pl.BlockSpec((r, h), lambda i: (i, 0)),
            pl.BlockSpec((n, h), lambda i: (0, 0)),
            pl.BlockSpec((r, 1), lambda i: (i, 0)),
            pl.BlockSpec((n, 1), lambda i: (0, 0)),
        ],
        out_specs=pl.BlockSpec(memory_space=pltpu.SMEM),
        out_shape=jax.ShapeDtypeStruct((2,), jnp.float32),
        scratch_shapes=[
            pltpu.SMEM((2,), jnp.float32),
            pltpu.VMEM((1, h), jnp.float32),
            pltpu.VMEM((n, h), jnp.float32),
        ],
        compiler_params=pltpu.CompilerParams(
            dimension_semantics=("arbitrary",),
        ),
    )(hs, ht, ms, mt)
    return (out[0], out[1])


# speedup vs baseline: 3.2105x; 1.0730x over previous
import jax
import jax.numpy as jnp
from jax.experimental import pallas as pl
from jax.experimental.pallas import tpu as pltpu

_TEMPERATURE = 0.07
_ROW_BLOCK = 512


def _tcl_body(hs_ref, ht_ref, ms_ref, mt_ref, out_ref, acc_ref, carry_ref,
              htn_ref):
    i = pl.program_id(0)
    nb = pl.num_programs(0)
    n = ht_ref.shape[0]
    r = hs_ref.shape[0]

    @pl.when(i == 0)
    def _prep():
        ht = ht_ref[...] * mt_ref[...]
        tinv = jax.lax.rsqrt(
            jnp.maximum(jnp.sum(ht * ht, axis=1, keepdims=True), 1e-24))
        htn_ref[...] = ht * tinv

    htn = htn_ref[...]

    hs = hs_ref[...] * ms_ref[...]
    sinv = jax.lax.rsqrt(
        jnp.maximum(jnp.sum(hs * hs, axis=1, keepdims=True), 1e-24))
    hsn = hs * (sinv * (1.4426950408889634 / _TEMPERATURE))

    sim = jax.lax.dot_general(hsn, htn, (((1,), (1,)), ((), ())),
                              preferred_element_type=jnp.float32)

    m = jnp.max(sim, axis=1, keepdims=True)

    log_s = jnp.log2(jnp.sum(jnp.exp2(sim), axis=1)) - m[:, 0]

    onehot = (sim == m).astype(jnp.float32)
    g = jax.lax.dot_general(onehot, htn, (((1,), (0,)), ((), ())),
                            preferred_element_type=jnp.float32)

    nn_step = jnp.sum(g[: r - 1, :] * g[1:, :])

    @pl.when(i == 0)
    def _init():
        acc_ref[0] = 0.0
        acc_ref[1] = 0.0

    @pl.when(i > 0)
    def _boundary():
        acc_ref[1] += jnp.sum(carry_ref[0, :] * g[0, :])

    acc_ref[0] += jnp.sum(log_s)
    acc_ref[1] += nn_step
    carry_ref[0, :] = g[r - 1, :]

    @pl.when(i == nb - 1)
    def _emit():
        out_ref[0] = acc_ref[0] * (0.6931471805599453 / n)
        out_ref[1] = 1.0 - acc_ref[1] / (n - 1)


def kernel(h_source, h_target, src_mask, tgt_mask):
    b, t, h = h_source.shape
    n = b * t
    r = _ROW_BLOCK
    hs = h_source.reshape(n, h).astype(jnp.float32)
    ht = h_target.reshape(n, h).astype(jnp.float32)
    ms = src_mask.reshape(n, 1).astype(jnp.float32)
    mt = tgt_mask.reshape(n, 1).astype(jnp.float32)

    out = pl.pallas_call(
        _tcl_body,
        grid=(n // r,),
        in_specs=[
            pl.BlockSpec((r, h), lambda i: (i, 0)),
            pl.BlockSpec((n, h), lambda i: (0, 0)),
            pl.BlockSpec((r, 1), lambda i: (i, 0)),
            pl.BlockSpec((n, 1), lambda i: (0, 0)),
        ],
        out_specs=pl.BlockSpec(memory_space=pltpu.SMEM),
        out_shape=jax.ShapeDtypeStruct((2,), jnp.float32),
        scratch_shapes=[
            pltpu.SMEM((2,), jnp.float32),
            pltpu.VMEM((1, h), jnp.float32),
            pltpu.VMEM((n, h), jnp.float32),
        ],
        compiler_params=pltpu.CompilerParams(
            dimension_semantics=("arbitrary",),
        ),
    )(hs, ht, ms, mt)
    return (out[0], out[1])
